# Initial kernel scaffold; baseline (speedup 1.0000x reference)
import jax
import jax.numpy as jnp
from jax.experimental import pallas as pl


def kernel(x, edge_index, W_gcn, b_gcn, W_ih, W_hh, b_ih, b_hh, W_fc, b_fc):
    raise NotImplementedError("write your pallas kernel here")


# speedup vs baseline: 6.1568x; 6.1568x over previous
import functools

import jax
import jax.numpy as jnp
from jax import lax
from jax.experimental import pallas as pl
from jax.experimental.pallas import tpu as pltpu
from jax.experimental.pallas import tpu_sc as plsc

F32 = jnp.float32


def _round_up(v, m):
    return ((v + m - 1) // m) * m


def _sc_hist(NP, EP):
    n_chunks = EP // (32 * 128)
    rows_per_tile = NP // 16
    mesh = plsc.VectorSubcoreMesh(core_axis_name="c", subcore_axis_name="s")

    @functools.partial(
        pl.kernel,
        out_type=jax.ShapeDtypeStruct((2 * NP, 128), F32),
        mesh=mesh,
        scratch_types=[
            pltpu.VMEM((n_chunks, 128), jnp.int32),
            pltpu.VMEM((128, 128), F32),
            pltpu.VMEM_SHARED((NP, 128), F32),
            pltpu.SemaphoreType.DMA,
        ],
    )
    def hist(dst_hbm, ones_hbm, zeros_hbm, deg_out, dst_v, ones_v, deg_sh, sem):
        c = lax.axis_index("c")
        s = lax.axis_index("s")
        wid = c * 16 + s
        pltpu.sync_copy(zeros_hbm, deg_sh.at[pl.ds(s * rows_per_tile, rows_per_tile)])
        pltpu.sync_copy(ones_hbm, ones_v)
        pltpu.sync_copy(dst_hbm.at[pl.ds(wid * n_chunks, n_chunks)], dst_v)
        plsc.subcore_barrier()

        def body(j, carry):
            pltpu.sync_copy(ones_v, deg_sh.at[dst_v.at[j]], add=True)
            return carry

        lax.fori_loop(0, n_chunks, body, 0)
        plsc.subcore_barrier()
        pltpu.sync_copy(
            deg_sh.at[pl.ds(s * rows_per_tile, rows_per_tile)],
            deg_out.at[pl.ds(c * NP + s * rows_per_tile, rows_per_tile)],
        )

    return hist


def _sc_agg(NP, EP):
    n_chunks = EP // (16 * 128)
    rows_per_tile = NP // 16
    mesh = plsc.VectorSubcoreMesh(core_axis_name="c", subcore_axis_name="s")

    @functools.partial(
        pl.kernel,
        out_type=jax.ShapeDtypeStruct((2 * NP, 128), F32),
        mesh=mesh,
        scratch_types=[
            pltpu.VMEM((n_chunks, 128), jnp.int32),
            pltpu.VMEM((n_chunks, 128), jnp.int32),
            pltpu.VMEM((128, 128), F32),
            pltpu.VMEM_SHARED((NP, 128), F32),
            pltpu.SemaphoreType.DMA,
        ],
    )
    def agg(y2_hbm, src2_hbm, dst_hbm, zeros_hbm, agg_out,
            src_v, dst_v, rows_v, agg_sh, sem):
        c = lax.axis_index("c")
        s = lax.axis_index("s")
        pltpu.sync_copy(zeros_hbm, agg_sh.at[pl.ds(s * rows_per_tile, rows_per_tile)])
        pltpu.sync_copy(
            src2_hbm.at[pl.ds((c * 16 + s) * n_chunks, n_chunks)], src_v)
        pltpu.sync_copy(dst_hbm.at[pl.ds(s * n_chunks, n_chunks)], dst_v)
        plsc.subcore_barrier()

        def body(j, carry):
            pltpu.async_copy(y2_hbm.at[src_v.at[j]], rows_v, sem).wait()
            pltpu.sync_copy(rows_v, agg_sh.at[dst_v.at[j]], add=True)
            return carry

        lax.fori_loop(0, n_chunks, body, 0)
        plsc.subcore_barrier()
        pltpu.sync_copy(
            agg_sh.at[pl.ds(s * rows_per_tile, rows_per_tile)],
            agg_out.at[pl.ds(c * NP + s * rows_per_tile, rows_per_tile)],
        )

    return agg


def _tc_a(NP, D, R, interpret=False):
    nr = NP // R

    def body(x_ref, w_ref, degA_ref, degB_ref, y_ref):
        deg = 0.0078125 * jnp.sum(
            degA_ref[...] + degB_ref[...], axis=1, keepdims=True) + 1.0
        dinv = lax.rsqrt(deg)
        xw = jnp.dot(x_ref[...], w_ref[...], preferred_element_type=F32)
        y_ref[...] = xw * dinv

    return pl.pallas_call(
        body,
        grid=(nr, 2),
        in_specs=[
            pl.BlockSpec((R, D), lambda r, c: (r, 0)),
            pl.BlockSpec((D, 128), lambda r, c: (0, c)),
            pl.BlockSpec((R, 128), lambda r, c: (r, 0)),
            pl.BlockSpec((R, 128), lambda r, c: (r + NP // R, 0)),
        ],
        out_specs=pl.BlockSpec((R, 128), lambda r, c: (c * nr + r, 0)),
        out_shape=jax.ShapeDtypeStruct((2 * NP, 128), F32),
        interpret=interpret,
    )


def _tc_b(NP, H, R, interpret=False):
    nr = NP // R

    def body(aggL_ref, aggH_ref, yL_ref, yH_ref, degA_ref, degB_ref,
             bgcn_ref, wih_ref, bih_ref, bhh_ref, gi_ref):
        deg = 0.0078125 * jnp.sum(
            degA_ref[...] + degB_ref[...], axis=1, keepdims=True) + 1.0
        dinv = lax.rsqrt(deg)
        hL = (aggL_ref[...] + yL_ref[...]) * dinv
        hH = (aggH_ref[...] + yH_ref[...]) * dinv
        h = jnp.concatenate([hL, hH], axis=1) + bgcn_ref[...]
        h = jnp.maximum(h, 0.0)
        col = lax.broadcasted_iota(jnp.int32, (1, 3 * H), 1)
        bias = bih_ref[...] + jnp.where(col < 2 * H, bhh_ref[...], 0.0)
        gi_ref[...] = (
            jnp.dot(h, wih_ref[...], preferred_element_type=F32) + bias)

    half = pl.BlockSpec((R, 128), lambda r: (r, 0))
    halfH = pl.BlockSpec((R, 128), lambda r: (r + nr, 0))
    return pl.pallas_call(
        body,
        grid=(nr,),
        in_specs=[
            half, halfH, half, halfH,
            pl.BlockSpec((R, 128), lambda r: (r, 0)),
            pl.BlockSpec((R, 128), lambda r: (r + nr, 0)),
            pl.BlockSpec((1, H), lambda r: (0, 0)),
            pl.BlockSpec((H, 3 * H), lambda r: (0, 0)),
            pl.BlockSpec((1, 3 * H), lambda r: (0, 0)),
            pl.BlockSpec((1, 3 * H), lambda r: (0, 0)),
        ],
        out_specs=pl.BlockSpec((R, 3 * H), lambda r: (r, 0)),
        out_shape=jax.ShapeDtypeStruct((NP, 3 * H), F32),
        interpret=interpret,
    )


def _tc_scan(NP, T, H, CHUNK, interpret=False):
    ng = T // CHUNK

    def body(gi_ref, whh_ref, bhh_ref, wfc_ref, bfc_ref, out_ref, h_scr):
        t = pl.program_id(0)

        @pl.when(t == 0)
        def _():
            h_scr[...] = jnp.zeros_like(h_scr)

        bn = bhh_ref[:, 2 * H:]

        def step(i, h):
            gi = gi_ref[pl.ds(i, 1), :]
            gh = jnp.dot(h, whh_ref[...], preferred_element_type=F32)
            r = jax.nn.sigmoid(gi[:, :H] + gh[:, :H])
            z = jax.nn.sigmoid(gi[:, H:2 * H] + gh[:, H:2 * H])
            n = jnp.tanh(gi[:, 2 * H:] + r * (gh[:, 2 * H:] + bn))
            return n + z * (h - n)

        h = lax.fori_loop(0, CHUNK, step, h_scr[...])
        h_scr[...] = h

        @pl.when(t == ng - 1)
        def _():
            out_ref[...] = (
                jnp.dot(h, wfc_ref[...], preferred_element_type=F32)
                + bfc_ref[...])

    return pl.pallas_call(
        body,
        grid=(ng,),
        in_specs=[
            pl.BlockSpec((CHUNK, 3 * H), lambda t: (t, 0)),
            pl.BlockSpec((H, 3 * H), lambda t: (0, 0)),
            pl.BlockSpec((1, 3 * H), lambda t: (0, 0)),
            pl.BlockSpec((H, 128), lambda t: (0, 0)),
            pl.BlockSpec((1, 128), lambda t: (0, 0)),
        ],
        out_specs=pl.BlockSpec((1, 128), lambda t: (0, 0)),
        out_shape=jax.ShapeDtypeStruct((1, 128), F32),
        scratch_shapes=[pltpu.VMEM((1, H), F32)],
        interpret=interpret,
    )


def kernel(x, edge_index, W_gcn, b_gcn, W_ih, W_hh, b_ih, b_hh, W_fc, b_fc):
    N, D = x.shape
    H = W_hh.shape[0]
    E = edge_index.shape[1]
    NP = _round_up(N, 2048)
    EP = _round_up(E, 32 * 128)
    R = 512

    x_pad = jnp.pad(x, ((0, NP - N), (0, 0)))
    src = edge_index[0]
    dst = edge_index[1]
    pad_e = EP - E
    pad_idx = jnp.full((pad_e,), N, jnp.int32)
    src_p = jnp.concatenate([src, pad_idx])
    dst_p = jnp.concatenate([dst, pad_idx])
    src2 = jnp.concatenate([src_p, src_p + NP]).reshape(2 * EP // 128, 128)
    dst2d = dst_p.reshape(EP // 128, 128)

    rows_per_tile = NP // 16
    ones_tile = jnp.ones((128, 128), F32)
    zeros128 = jnp.zeros((rows_per_tile, 128), F32)

    deg2 = _sc_hist(NP, EP)(dst2d, ones_tile, zeros128)
    y2 = _tc_a(NP, D, R)(x_pad, W_gcn, deg2, deg2)
    agg2 = _sc_agg(NP, EP)(y2, src2, dst2d, zeros128)
    gi = _tc_b(NP, H, R)(
        agg2, agg2, y2, y2, deg2, deg2,
        b_gcn.reshape(1, H), W_ih,
        b_ih.reshape(1, 3 * H), b_hh.reshape(1, 3 * H))

    wfc_pad = jnp.pad(W_fc, ((0, 0), (0, 128 - W_fc.shape[1])))
    bfc_pad = jnp.pad(b_fc, (0, 128 - b_fc.shape[0])).reshape(1, 128)
    out = _tc_scan(NP, N, H, 1000)(
        gi, W_hh, b_hh.reshape(1, 3 * H), wfc_pad, bfc_pad)
    return out[:, :1]


# speedup vs baseline: 6.2941x; 1.0223x over previous
import functools

import jax
import jax.numpy as jnp
from jax import lax
from jax.experimental import pallas as pl
from jax.experimental.pallas import tpu as pltpu
from jax.experimental.pallas import tpu_sc as plsc

F32 = jnp.float32


def _round_up(v, m):
    return ((v + m - 1) // m) * m


def _sc_hist(NP, EP):
    n_chunks = EP // (32 * 128)
    rows_per_tile = NP // 16
    mesh = plsc.VectorSubcoreMesh(core_axis_name="c", subcore_axis_name="s")

    @functools.partial(
        pl.kernel,
        out_type=jax.ShapeDtypeStruct((2 * NP, 128), F32),
        mesh=mesh,
        scratch_types=[
            pltpu.VMEM((n_chunks, 128), jnp.int32),
            pltpu.VMEM((128, 128), F32),
            pltpu.VMEM_SHARED((NP, 128), F32),
            pltpu.SemaphoreType.DMA,
        ],
    )
    def hist(dst_hbm, ones_hbm, zeros_hbm, deg_out, dst_v, ones_v, deg_sh, sem):
        c = lax.axis_index("c")
        s = lax.axis_index("s")
        wid = c * 16 + s
        pltpu.sync_copy(zeros_hbm, deg_sh.at[pl.ds(s * rows_per_tile, rows_per_tile)])
        pltpu.sync_copy(ones_hbm, ones_v)
        pltpu.sync_copy(dst_hbm.at[pl.ds(wid * n_chunks, n_chunks)], dst_v)
        plsc.subcore_barrier()

        def body(j, carry):
            pltpu.sync_copy(ones_v, deg_sh.at[dst_v.at[j]], add=True)
            return carry

        lax.fori_loop(0, n_chunks, body, 0)
        plsc.subcore_barrier()
        pltpu.sync_copy(
            deg_sh.at[pl.ds(s * rows_per_tile, rows_per_tile)],
            deg_out.at[pl.ds(c * NP + s * rows_per_tile, rows_per_tile)],
        )

    return hist


def _sc_agg(NP, EP):
    n_chunks = EP // (16 * 128)
    rows_per_tile = NP // 16
    mesh = plsc.VectorSubcoreMesh(core_axis_name="c", subcore_axis_name="s")

    @functools.partial(
        pl.kernel,
        out_type=jax.ShapeDtypeStruct((2 * NP, 128), F32),
        mesh=mesh,
        scratch_types=[
            pltpu.VMEM((n_chunks, 128), jnp.int32),
            pltpu.VMEM((n_chunks, 128), jnp.int32),
            pltpu.VMEM((128, 128), F32),
            pltpu.VMEM_SHARED((NP, 128), F32),
            pltpu.SemaphoreType.DMA,
        ],
    )
    def agg(y2_hbm, src2_hbm, dst_hbm, zeros_hbm, agg_out,
            src_v, dst_v, rows_v, agg_sh, sem):
        c = lax.axis_index("c")
        s = lax.axis_index("s")
        pltpu.sync_copy(zeros_hbm, agg_sh.at[pl.ds(s * rows_per_tile, rows_per_tile)])
        pltpu.sync_copy(
            src2_hbm.at[pl.ds((c * 16 + s) * n_chunks, n_chunks)], src_v)
        pltpu.sync_copy(dst_hbm.at[pl.ds(s * n_chunks, n_chunks)], dst_v)
        plsc.subcore_barrier()

        def body(j, carry):
            pltpu.async_copy(y2_hbm.at[src_v.at[j]], rows_v, sem).wait()
            pltpu.sync_copy(rows_v, agg_sh.at[dst_v.at[j]], add=True)
            return carry

        lax.fori_loop(0, n_chunks, body, 0)
        plsc.subcore_barrier()
        pltpu.sync_copy(
            agg_sh.at[pl.ds(s * rows_per_tile, rows_per_tile)],
            agg_out.at[pl.ds(c * NP + s * rows_per_tile, rows_per_tile)],
        )

    return agg


def _tc_a(NP, D, R, interpret=False):
    nr = NP // R

    def body(x_ref, w_ref, degA_ref, degB_ref, y_ref):
        deg = 0.0078125 * jnp.sum(
            degA_ref[...] + degB_ref[...], axis=1, keepdims=True) + 1.0
        dinv = lax.rsqrt(deg)
        xw = jnp.dot(x_ref[...], w_ref[...], preferred_element_type=F32)
        y_ref[...] = xw * dinv

    return pl.pallas_call(
        body,
        grid=(nr, 2),
        in_specs=[
            pl.BlockSpec((R, D), lambda r, c: (r, 0)),
            pl.BlockSpec((D, 128), lambda r, c: (0, c)),
            pl.BlockSpec((R, 128), lambda r, c: (r, 0)),
            pl.BlockSpec((R, 128), lambda r, c: (r + NP // R, 0)),
        ],
        out_specs=pl.BlockSpec((R, 128), lambda r, c: (c * nr + r, 0)),
        out_shape=jax.ShapeDtypeStruct((2 * NP, 128), F32),
        interpret=interpret,
    )


def _tc_b(NP, H, R, interpret=False):
    nr = NP // R

    def body(aggL_ref, aggH_ref, yL_ref, yH_ref, degA_ref, degB_ref,
             bgcn_ref, wih_ref, bih_ref, bhh_ref, gi_ref):
        deg = 0.0078125 * jnp.sum(
            degA_ref[...] + degB_ref[...], axis=1, keepdims=True) + 1.0
        dinv = lax.rsqrt(deg)
        hL = (aggL_ref[...] + yL_ref[...]) * dinv
        hH = (aggH_ref[...] + yH_ref[...]) * dinv
        h = jnp.concatenate([hL, hH], axis=1) + bgcn_ref[...]
        h = jnp.maximum(h, 0.0)
        col = lax.broadcasted_iota(jnp.int32, (1, 3 * H), 1)
        bias = bih_ref[...] + jnp.where(col < 2 * H, bhh_ref[...], 0.0)
        gi_ref[...] = (
            jnp.dot(h, wih_ref[...], preferred_element_type=F32) + bias)

    half = pl.BlockSpec((R, 128), lambda r: (r, 0))
    halfH = pl.BlockSpec((R, 128), lambda r: (r + nr, 0))
    return pl.pallas_call(
        body,
        grid=(nr,),
        in_specs=[
            half, halfH, half, halfH,
            pl.BlockSpec((R, 128), lambda r: (r, 0)),
            pl.BlockSpec((R, 128), lambda r: (r + nr, 0)),
            pl.BlockSpec((1, H), lambda r: (0, 0)),
            pl.BlockSpec((H, 3 * H), lambda r: (0, 0)),
            pl.BlockSpec((1, 3 * H), lambda r: (0, 0)),
            pl.BlockSpec((1, 3 * H), lambda r: (0, 0)),
        ],
        out_specs=pl.BlockSpec((R, 3 * H), lambda r: (r, 0)),
        out_shape=jax.ShapeDtypeStruct((NP, 3 * H), F32),
        interpret=interpret,
    )


def _tc_scan(NP, T, H, CHUNK, interpret=False):
    ng = T // CHUNK

    def body(gi_ref, whh_ref, bhh_ref, wfc_ref, bfc_ref, out_ref, h_scr):
        t = pl.program_id(0)

        @pl.when(t == 0)
        def _():
            h_scr[...] = jnp.zeros_like(h_scr)

        bn = bhh_ref[:, 2 * H:]

        def step(i, h):
            gi = gi_ref[pl.ds(i, 1), :]
            gh = jnp.dot(h.astype(whh_ref.dtype), whh_ref[...],
                         preferred_element_type=F32)
            r = jax.nn.sigmoid(gi[:, :H] + gh[:, :H])
            z = jax.nn.sigmoid(gi[:, H:2 * H] + gh[:, H:2 * H])
            n = jnp.tanh(gi[:, 2 * H:] + r * (gh[:, 2 * H:] + bn))
            return n + z * (h - n)

        h = lax.fori_loop(0, CHUNK, step, h_scr[...])
        h_scr[...] = h

        @pl.when(t == ng - 1)
        def _():
            out_ref[...] = (
                jnp.dot(h, wfc_ref[...], preferred_element_type=F32)
                + bfc_ref[...])

    return pl.pallas_call(
        body,
        grid=(ng,),
        in_specs=[
            pl.BlockSpec((CHUNK, 3 * H), lambda t: (t, 0)),
            pl.BlockSpec((H, 3 * H), lambda t: (0, 0)),
            pl.BlockSpec((1, 3 * H), lambda t: (0, 0)),
            pl.BlockSpec((H, 128), lambda t: (0, 0)),
            pl.BlockSpec((1, 128), lambda t: (0, 0)),
        ],
        out_specs=pl.BlockSpec((1, 128), lambda t: (0, 0)),
        out_shape=jax.ShapeDtypeStruct((1, 128), F32),
        scratch_shapes=[pltpu.VMEM((1, H), F32)],
        interpret=interpret,
    )


def kernel(x, edge_index, W_gcn, b_gcn, W_ih, W_hh, b_ih, b_hh, W_fc, b_fc):
    N, D = x.shape
    H = W_hh.shape[0]
    E = edge_index.shape[1]
    NP = _round_up(N, 2048)
    EP = _round_up(E, 32 * 128)
    R = 512

    x_pad = jnp.pad(x, ((0, NP - N), (0, 0)))
    src = edge_index[0]
    dst = edge_index[1]
    pad_e = EP - E
    pad_idx = jnp.full((pad_e,), N, jnp.int32)
    src_p = jnp.concatenate([src, pad_idx])
    dst_p = jnp.concatenate([dst, pad_idx])
    src2 = jnp.concatenate([src_p, src_p + NP]).reshape(2 * EP // 128, 128)
    dst2d = dst_p.reshape(EP // 128, 128)

    rows_per_tile = NP // 16
    ones_tile = jnp.ones((128, 128), F32)
    zeros128 = jnp.zeros((rows_per_tile, 128), F32)

    deg2 = _sc_hist(NP, EP)(dst2d, ones_tile, zeros128)
    y2 = _tc_a(NP, D, R)(x_pad, W_gcn, deg2, deg2)
    agg2 = _sc_agg(NP, EP)(y2, src2, dst2d, zeros128)
    gi = _tc_b(NP, H, R)(
        agg2, agg2, y2, y2, deg2, deg2,
        b_gcn.reshape(1, H), W_ih,
        b_ih.reshape(1, 3 * H), b_hh.reshape(1, 3 * H))

    wfc_pad = jnp.pad(W_fc, ((0, 0), (0, 128 - W_fc.shape[1])))
    bfc_pad = jnp.pad(b_fc, (0, 128 - b_fc.shape[0])).reshape(1, 128)
    out = _tc_scan(NP, N, H, 1000)(
        gi, W_hh.astype(jnp.bfloat16), b_hh.reshape(1, 3 * H),
        wfc_pad, bfc_pad)
    return out[:, :1]


# speedup vs baseline: 7.2844x; 1.1573x over previous
import functools

import jax
import jax.numpy as jnp
from jax import lax
from jax.experimental import pallas as pl
from jax.experimental.pallas import tpu as pltpu
from jax.experimental.pallas import tpu_sc as plsc

F32 = jnp.float32


def _round_up(v, m):
    return ((v + m - 1) // m) * m


def _sc_hist(NP, EP):
    n_chunks = EP // (32 * 128)
    rows_per_tile = NP // 16
    mesh = plsc.VectorSubcoreMesh(core_axis_name="c", subcore_axis_name="s")

    @functools.partial(
        pl.kernel,
        out_type=jax.ShapeDtypeStruct((2 * NP, 128), F32),
        mesh=mesh,
        scratch_types=[
            pltpu.VMEM((n_chunks, 128), jnp.int32),
            pltpu.VMEM((128, 128), F32),
            pltpu.VMEM_SHARED((NP, 128), F32),
            pltpu.SemaphoreType.DMA,
        ],
    )
    def hist(dst_hbm, ones_hbm, zeros_hbm, deg_out, dst_v, ones_v, deg_sh, sem):
        c = lax.axis_index("c")
        s = lax.axis_index("s")
        wid = c * 16 + s
        pltpu.sync_copy(zeros_hbm, deg_sh.at[pl.ds(s * rows_per_tile, rows_per_tile)])
        pltpu.sync_copy(ones_hbm, ones_v)
        pltpu.sync_copy(dst_hbm.at[pl.ds(wid * n_chunks, n_chunks)], dst_v)
        plsc.subcore_barrier()

        def body(j, carry):
            pltpu.sync_copy(ones_v, deg_sh.at[dst_v.at[j]], add=True)
            return carry

        lax.fori_loop(0, n_chunks, body, 0)
        plsc.subcore_barrier()
        pltpu.sync_copy(
            deg_sh.at[pl.ds(s * rows_per_tile, rows_per_tile)],
            deg_out.at[pl.ds(c * NP + s * rows_per_tile, rows_per_tile)],
        )

    return hist


def _sc_agg(NP, EP):
    n_chunks = EP // (16 * 128)
    rows_per_tile = NP // 16
    mesh = plsc.VectorSubcoreMesh(core_axis_name="c", subcore_axis_name="s")

    @functools.partial(
        pl.kernel,
        out_type=jax.ShapeDtypeStruct((2 * NP, 128), F32),
        mesh=mesh,
        scratch_types=[
            pltpu.VMEM((n_chunks, 128), jnp.int32),
            pltpu.VMEM((n_chunks, 128), jnp.int32),
            pltpu.VMEM((128, 128), F32),
            pltpu.VMEM_SHARED((NP, 128), F32),
            pltpu.SemaphoreType.DMA,
        ],
    )
    def agg(y2_hbm, src2_hbm, dst_hbm, zeros_hbm, agg_out,
            src_v, dst_v, rows_v, agg_sh, sem):
        c = lax.axis_index("c")
        s = lax.axis_index("s")
        pltpu.sync_copy(zeros_hbm, agg_sh.at[pl.ds(s * rows_per_tile, rows_per_tile)])
        pltpu.sync_copy(
            src2_hbm.at[pl.ds((c * 16 + s) * n_chunks, n_chunks)], src_v)
        pltpu.sync_copy(dst_hbm.at[pl.ds(s * n_chunks, n_chunks)], dst_v)
        plsc.subcore_barrier()

        def body(j, carry):
            pltpu.async_copy(y2_hbm.at[src_v.at[j]], rows_v, sem).wait()
            pltpu.sync_copy(rows_v, agg_sh.at[dst_v.at[j]], add=True)
            return carry

        lax.fori_loop(0, n_chunks, body, 0)
        plsc.subcore_barrier()
        pltpu.sync_copy(
            agg_sh.at[pl.ds(s * rows_per_tile, rows_per_tile)],
            agg_out.at[pl.ds(c * NP + s * rows_per_tile, rows_per_tile)],
        )

    return agg


def _tc_a(NP, D, R, interpret=False):
    nr = NP // R

    def body(x_ref, w_ref, degA_ref, degB_ref, y_ref):
        deg = 0.0078125 * jnp.sum(
            degA_ref[...] + degB_ref[...], axis=1, keepdims=True) + 1.0
        dinv = lax.rsqrt(deg)
        xw = jnp.dot(x_ref[...], w_ref[...], preferred_element_type=F32)
        y_ref[...] = xw * dinv

    return pl.pallas_call(
        body,
        grid=(nr, 2),
        in_specs=[
            pl.BlockSpec((R, D), lambda r, c: (r, 0)),
            pl.BlockSpec((D, 128), lambda r, c: (0, c)),
            pl.BlockSpec((R, 128), lambda r, c: (r, 0)),
            pl.BlockSpec((R, 128), lambda r, c: (r + NP // R, 0)),
        ],
        out_specs=pl.BlockSpec((R, 128), lambda r, c: (c * nr + r, 0)),
        out_shape=jax.ShapeDtypeStruct((2 * NP, 128), F32),
        interpret=interpret,
    )


def _tc_b(NP, H, R, interpret=False):
    nr = NP // R

    def body(aggL_ref, aggH_ref, yL_ref, yH_ref, degA_ref, degB_ref,
             bgcn_ref, wih_ref, bih_ref, bhh_ref, gi_ref):
        deg = 0.0078125 * jnp.sum(
            degA_ref[...] + degB_ref[...], axis=1, keepdims=True) + 1.0
        dinv = lax.rsqrt(deg)
        hL = (aggL_ref[...] + yL_ref[...]) * dinv
        hH = (aggH_ref[...] + yH_ref[...]) * dinv
        h = jnp.concatenate([hL, hH], axis=1) + bgcn_ref[...]
        h = jnp.maximum(h, 0.0)
        col = lax.broadcasted_iota(jnp.int32, (1, 3 * H), 1)
        bias = bih_ref[...] + jnp.where(col < 2 * H, bhh_ref[...], 0.0)
        gi_ref[...] = (
            jnp.dot(h, wih_ref[...], preferred_element_type=F32) + bias)

    half = pl.BlockSpec((R, 128), lambda r: (r, 0))
    halfH = pl.BlockSpec((R, 128), lambda r: (r + nr, 0))
    return pl.pallas_call(
        body,
        grid=(nr,),
        in_specs=[
            half, halfH, half, halfH,
            pl.BlockSpec((R, 128), lambda r: (r, 0)),
            pl.BlockSpec((R, 128), lambda r: (r + nr, 0)),
            pl.BlockSpec((1, H), lambda r: (0, 0)),
            pl.BlockSpec((H, 3 * H), lambda r: (0, 0)),
            pl.BlockSpec((1, 3 * H), lambda r: (0, 0)),
            pl.BlockSpec((1, 3 * H), lambda r: (0, 0)),
        ],
        out_specs=pl.BlockSpec((R, 3 * H), lambda r: (r, 0)),
        out_shape=jax.ShapeDtypeStruct((NP, 3 * H), F32),
        interpret=interpret,
    )


def _tc_scan(NP, T, H, CHUNK, interpret=False):
    ng = T // CHUNK

    def body(gi_ref, w1_ref, w2_ref, bhh_ref, wfc_ref, bfc_ref, out_ref,
             h_scr):
        t = pl.program_id(0)

        @pl.when(t == 0)
        def _():
            h_scr[...] = jnp.zeros_like(h_scr)

        bn = bhh_ref[:, 2 * H:]

        hw = 3 * H // 2

        def block(b, h):
            base = pl.multiple_of(b * 8, 8)
            tile = gi_ref[pl.ds(base, 8), :]
            for k in range(8):
                gi = tile[k:k + 1, :]
                hb = h.astype(w1_ref.dtype)
                gh1 = jnp.dot(hb, w1_ref[...], preferred_element_type=F32)
                gh2 = jnp.dot(hb, w2_ref[...], preferred_element_type=F32)
                r = 0.5 + 0.5 * jnp.tanh(0.5 * (gi[:, :H] + gh1[:, :H]))
                z = 0.5 + 0.5 * jnp.tanh(0.5 * (
                    gi[:, H:2 * H]
                    + jnp.concatenate([gh1[:, H:], gh2[:, :2 * H - hw]], 1)))
                n = jnp.tanh(gi[:, 2 * H:] + r * (gh2[:, 2 * H - hw:] + bn))
                h = n + z * (h - n)
            return h

        h = lax.fori_loop(0, CHUNK // 8, block, h_scr[...])
        h_scr[...] = h

        @pl.when(t == ng - 1)
        def _():
            out_ref[...] = (
                jnp.dot(h, wfc_ref[...], preferred_element_type=F32)
                + bfc_ref[...])

    return pl.pallas_call(
        body,
        grid=(ng,),
        in_specs=[
            pl.BlockSpec((CHUNK, 3 * H), lambda t: (t, 0)),
            pl.BlockSpec((H, 3 * H // 2), lambda t: (0, 0)),
            pl.BlockSpec((H, 3 * H // 2), lambda t: (0, 0)),
            pl.BlockSpec((1, 3 * H), lambda t: (0, 0)),
            pl.BlockSpec((H, 128), lambda t: (0, 0)),
            pl.BlockSpec((1, 128), lambda t: (0, 0)),
        ],
        out_specs=pl.BlockSpec((1, 128), lambda t: (0, 0)),
        out_shape=jax.ShapeDtypeStruct((1, 128), F32),
        scratch_shapes=[pltpu.VMEM((1, H), F32)],
        interpret=interpret,
    )


def kernel(x, edge_index, W_gcn, b_gcn, W_ih, W_hh, b_ih, b_hh, W_fc, b_fc):
    N, D = x.shape
    H = W_hh.shape[0]
    E = edge_index.shape[1]
    NP = _round_up(N, 2048)
    EP = _round_up(E, 32 * 128)
    R = 512

    x_pad = jnp.pad(x, ((0, NP - N), (0, 0)))
    src = edge_index[0]
    dst = edge_index[1]
    pad_e = EP - E
    pad_idx = jnp.full((pad_e,), N, jnp.int32)
    src_p = jnp.concatenate([src, pad_idx])
    dst_p = jnp.concatenate([dst, pad_idx])
    src2 = jnp.concatenate([src_p, src_p + NP]).reshape(2 * EP // 128, 128)
    dst2d = dst_p.reshape(EP // 128, 128)

    rows_per_tile = NP // 16
    ones_tile = jnp.ones((128, 128), F32)
    zeros128 = jnp.zeros((rows_per_tile, 128), F32)

    deg2 = _sc_hist(NP, EP)(dst2d, ones_tile, zeros128)
    y2 = _tc_a(NP, D, R)(x_pad, W_gcn, deg2, deg2)
    agg2 = _sc_agg(NP, EP)(y2, src2, dst2d, zeros128)
    gi = _tc_b(NP, H, R)(
        agg2, agg2, y2, y2, deg2, deg2,
        b_gcn.reshape(1, H), W_ih,
        b_ih.reshape(1, 3 * H), b_hh.reshape(1, 3 * H))

    wfc_pad = jnp.pad(W_fc, ((0, 0), (0, 128 - W_fc.shape[1])))
    bfc_pad = jnp.pad(b_fc, (0, 128 - b_fc.shape[0])).reshape(1, 128)
    whh_bf = W_hh.astype(jnp.bfloat16)
    hw = 3 * H // 2
    out = _tc_scan(NP, N, H, 1000)(
        gi, whh_bf[:, :hw], whh_bf[:, hw:], b_hh.reshape(1, 3 * H),
        wfc_pad, bfc_pad)
    return out[:, :1]
